# baseline (speedup 1.0000x reference)
import functools

import jax
import jax.numpy as jnp
from jax import lax
from jax.experimental import pallas as pl
from jax.experimental.pallas import tpu as pltpu

B, S, D, FF, E, R = 1, 2048, 768, 3072, 8, 32
ER = E * R
SCALING = 16.0 / 32.0
NF = 8
FC = FF // NF

_DN_T = (((1,), (1,)), ((), ()))


def _dott(a, b):
    return lax.dot_general(a, b, _DN_T, preferred_element_type=jnp.float32)


def _dot(a, b):
    return jnp.dot(a, b, preferred_element_type=jnp.float32)


def _fused_kernel(x_ref, wr_ref, rb_ref,
                  w1_ref, b1_ref, w2_ref, b2_ref,
                  a1_ref, a1b_ref, bm1_ref, bb1_ref,
                  a2_ref, a2b_ref, bm2_ref, bb2_ref,
                  out_ref, routing_ref, choice_ref,
                  hw_s, rexp_s, h2_s, o2_s):
    f32 = jnp.float32
    f = pl.program_id(0)
    xt = x_ref[...]

    @pl.when(f == 0)
    def _first():
        logits = _dott(xt, wr_ref[...]) + rb_ref[...]
        routing = jax.nn.softmax(logits, axis=-1)
        routing_ref[...] = routing

        mx = jnp.max(routing, axis=-1, keepdims=True)
        eq = routing == mx
        iot = lax.broadcasted_iota(jnp.int32, routing.shape, 1)
        idx = jnp.min(jnp.where(eq, iot, E), axis=-1, keepdims=True)
        choice_ref[...] = (iot == idx).astype(f32)

        col = lax.broadcasted_iota(jnp.int32, (E, ER), 1) // R
        row = lax.broadcasted_iota(jnp.int32, (E, ER), 0)
        expand = (col == row).astype(f32)
        r_exp = _dot(routing, expand)
        rexp_s[...] = r_exp

        hw_s[...] = (_dott(xt, a1_ref[...]) + a1b_ref[...]) * r_exp

    routing = routing_ref[...]

    lora1 = _dot(hw_s[...], bm1_ref[...]) + _dot(routing, bb1_ref[...])
    orig1 = _dott(xt, w1_ref[...]) + b1_ref[...]
    h1 = jax.nn.gelu(orig1 + SCALING * lora1)

    o2 = _dott(h1, w2_ref[...])
    h2 = _dott(h1, a2_ref[...])

    @pl.when(f == 0)
    def _init():
        o2_s[...] = o2
        h2_s[...] = h2

    @pl.when(f > 0)
    def _acc():
        o2_s[...] += o2
        h2_s[...] += h2

    @pl.when(f == NF - 1)
    def _last():
        hw2 = (h2_s[...] + a2b_ref[...]) * rexp_s[...]
        lora2 = _dot(hw2, bm2_ref[...]) + _dot(routing, bb2_ref[...])
        out_ref[...] = o2_s[...] + b2_ref[...] + SCALING * lora2


@functools.partial(jax.jit, static_argnames=())
def kernel(x, router_W, router_b, fc1_W, fc1_b, fc2_W, fc2_b,
           down_A, down_A_b, down_B, down_B_b,
           up_A, up_A_b, up_B, up_B_b):
    f32 = jnp.float32
    xs = x.reshape(S, D)
    rb = router_b.reshape(1, E)
    b1 = fc1_b.reshape(1, FF)
    b2 = fc2_b.reshape(1, D)
    a1 = down_A.reshape(ER, D)
    a1b = down_A_b.reshape(1, ER)
    bm1 = down_B.transpose(0, 2, 1).reshape(ER, FF)
    a2 = up_A.reshape(ER, FF)
    a2b = up_A_b.reshape(1, ER)
    bm2 = up_B.transpose(0, 2, 1).reshape(ER, D)

    grid = (NF,)
    full = lambda shape: pl.BlockSpec(shape, lambda f: (0,) * len(shape))

    out, routing, choice = pl.pallas_call(
        _fused_kernel,
        grid=grid,
        in_specs=[
            full((S, D)),
            full((E, D)), full((1, E)),
            pl.BlockSpec((FC, D), lambda f: (f, 0)),
            pl.BlockSpec((1, FC), lambda f: (0, f)),
            pl.BlockSpec((D, FC), lambda f: (0, f)),
            full((1, D)),
            full((ER, D)), full((1, ER)),
            pl.BlockSpec((ER, FC), lambda f: (0, f)),
            pl.BlockSpec((E, FC), lambda f: (0, f)),
            pl.BlockSpec((ER, FC), lambda f: (0, f)),
            full((1, ER)), full((ER, D)), full((E, D)),
        ],
        out_specs=[full((S, D)), full((S, E)), full((S, E))],
        out_shape=[
            jax.ShapeDtypeStruct((S, D), f32),
            jax.ShapeDtypeStruct((S, E), f32),
            jax.ShapeDtypeStruct((S, E), f32),
        ],
        scratch_shapes=[
            pltpu.VMEM((S, ER), f32),
            pltpu.VMEM((S, ER), f32),
            pltpu.VMEM((S, ER), f32),
            pltpu.VMEM((S, D), f32),
        ],
    )(xs, router_W, rb, fc1_W, b1, fc2_W, b2,
      a1, a1b, bm1, down_B_b, a2, a2b, bm2, up_B_b)

    return (out.reshape(B, S, D),
            (routing.reshape(B, S, E), choice.reshape(B, S, E)))


# speedup vs baseline: 1.2130x; 1.2130x over previous
import functools

import jax
import jax.numpy as jnp
from jax import lax
from jax.experimental import pallas as pl

B, S, D, FF, E, R = 1, 2048, 768, 3072, 8, 32
ER = E * R
SCALING = 16.0 / 32.0
TILE = 512

_DN_T = (((1,), (1,)), ((), ()))


def _dott(a, b):
    return lax.dot_general(a, b, _DN_T, preferred_element_type=jnp.float32)


def _fused_kernel(x_ref, wr_ref, rb_ref,
                  w1_ref, b1_ref, w2_ref, b2_ref,
                  a1_ref, a1b_ref, bm1_ref, bb1_ref,
                  a2_ref, a2b_ref, bm2_ref, bb2_ref,
                  out_ref, routing_ref, choice_ref):
    f32 = jnp.float32
    xt = x_ref[...]

    logits = _dott(xt, wr_ref[...]) + rb_ref[...]
    routing = jax.nn.softmax(logits, axis=-1)
    routing_ref[...] = routing

    mx = jnp.max(routing, axis=-1, keepdims=True)
    eq = routing == mx
    iot = lax.broadcasted_iota(jnp.int32, routing.shape, 1)
    idx = jnp.min(jnp.where(eq, iot, E), axis=-1, keepdims=True)
    choice_ref[...] = (iot == idx).astype(f32)

    col = lax.broadcasted_iota(jnp.int32, (E, ER), 1) // R
    row = lax.broadcasted_iota(jnp.int32, (E, ER), 0)
    expand = (col == row).astype(f32)
    r_exp = jnp.dot(routing, expand, preferred_element_type=f32)

    h = _dott(xt, a1_ref[...]) + a1b_ref[...]
    lora1 = (jnp.dot(h * r_exp, bm1_ref[...], preferred_element_type=f32)
             + jnp.dot(routing, bb1_ref[...], preferred_element_type=f32))
    orig1 = _dott(xt, w1_ref[...]) + b1_ref[...]
    h1 = jax.nn.gelu(orig1 + SCALING * lora1)

    h2 = _dott(h1, a2_ref[...]) + a2b_ref[...]
    lora2 = (jnp.dot(h2 * r_exp, bm2_ref[...], preferred_element_type=f32)
             + jnp.dot(routing, bb2_ref[...], preferred_element_type=f32))
    orig2 = _dott(h1, w2_ref[...]) + b2_ref[...]
    out_ref[...] = orig2 + SCALING * lora2


@functools.partial(jax.jit, static_argnames=())
def kernel(x, router_W, router_b, fc1_W, fc1_b, fc2_W, fc2_b,
           down_A, down_A_b, down_B, down_B_b,
           up_A, up_A_b, up_B, up_B_b):
    f32 = jnp.float32
    xs = x.reshape(S, D)
    rb = router_b.reshape(1, E)
    b1 = fc1_b.reshape(1, FF)
    b2 = fc2_b.reshape(1, D)
    a1 = down_A.reshape(ER, D)
    a1b = down_A_b.reshape(1, ER)
    bm1 = down_B.transpose(0, 2, 1).reshape(ER, FF)
    a2 = up_A.reshape(ER, FF)
    a2b = up_A_b.reshape(1, ER)
    bm2 = up_B.transpose(0, 2, 1).reshape(ER, D)

    grid = (S // TILE,)
    full = lambda shape: pl.BlockSpec(shape, lambda i: (0,) * len(shape))
    tok = lambda w: pl.BlockSpec((TILE, w), lambda i: (i, 0))

    out, routing, choice = pl.pallas_call(
        _fused_kernel,
        grid=grid,
        in_specs=[
            tok(D),
            full((E, D)), full((1, E)),
            full((FF, D)), full((1, FF)), full((D, FF)), full((1, D)),
            full((ER, D)), full((1, ER)), full((ER, FF)), full((E, FF)),
            full((ER, FF)), full((1, ER)), full((ER, D)), full((E, D)),
        ],
        out_specs=[tok(D), tok(E), tok(E)],
        out_shape=[
            jax.ShapeDtypeStruct((S, D), f32),
            jax.ShapeDtypeStruct((S, E), f32),
            jax.ShapeDtypeStruct((S, E), f32),
        ],
    )(xs, router_W, rb, fc1_W, b1, fc2_W, b2,
      a1, a1b, bm1, down_B_b, a2, a2b, bm2, up_B_b)

    return (out.reshape(B, S, D),
            (routing.reshape(B, S, E), choice.reshape(B, S, E)))
